# baseline (speedup 1.0000x reference)
import functools

import jax
import jax.numpy as jnp
from jax import lax
from jax.experimental import pallas as pl
from jax.experimental.pallas import tpu as pltpu
from jax.experimental.pallas import tpu_sc as plsc

N = 10000
D = 128
E = 320000
HALF = 64
NS = 16
CHUNK = 128
CPT = 160
NCHUNK = CPT * NS
E_PAD = NCHUNK * CHUNK
NBLK = 4
CPB = CPT // NBLK
ROWS_PT = N // NS
R_STEP = 125
R_ITER = ROWS_PT // R_STEP


NBUF = 4


def _agg_body(h_hbm, src_hbm, dst_hbm, out_hbm,
              h_sh, agg_sh, sidx_v, didx_v,
              buf0, buf1, buf2, buf3, gsems, ssems):
    c = lax.axis_index("c")
    s = lax.axis_index("s")
    c0 = c * HALF
    r0 = s * ROWS_PT
    bufs = (buf0, buf1, buf2, buf3)

    pltpu.sync_copy(h_hbm.at[pl.ds(r0, ROWS_PT), pl.ds(c0, HALF)],
                    h_sh.at[pl.ds(r0, ROWS_PT)])
    pltpu.sync_copy(h_hbm.at[pl.ds(r0, ROWS_PT), pl.ds(c0, HALF)],
                    agg_sh.at[pl.ds(r0, ROWS_PT)])
    plsc.subcore_barrier()

    def gather(j, t):
        return pltpu.async_copy(h_sh.at[sidx_v.at[j]], bufs[t], gsems.at[t])

    def gather_wait(j, t):
        pltpu.make_async_copy(h_sh.at[sidx_v.at[j]], bufs[t],
                              gsems.at[t]).wait()

    def scatter(j, t):
        return pltpu.async_copy(bufs[t], agg_sh.at[didx_v.at[j]],
                                ssems.at[t], add=True)

    def scatter_wait(j, t):
        pltpu.make_async_copy(bufs[t], agg_sh.at[didx_v.at[j]],
                              ssems.at[t]).wait()

    def blk(bi, carry):
        ch0 = s * CPT + bi * CPB
        pltpu.sync_copy(src_hbm.at[pl.ds(ch0, CPB)], sidx_v)
        pltpu.sync_copy(dst_hbm.at[pl.ds(ch0, CPB)], didx_v)

        for t in range(NBUF):
            gather(t, t)

        def body(q, carry2):
            j = NBUF * q
            for t in range(NBUF):
                gather_wait(j + t, t)
                scatter(j + t, t)
            for t in range(NBUF):
                scatter_wait(j + t, t)
                gather(j + NBUF + t, t)
            return carry2

        lax.fori_loop(0, CPB // NBUF - 1, body, 0)

        jl = CPB - NBUF
        for t in range(NBUF):
            gather_wait(jl + t, t)
            scatter(jl + t, t)
        for t in range(NBUF):
            scatter_wait(jl + t, t)
        return carry

    lax.fori_loop(0, NBLK, blk, 0)
    plsc.subcore_barrier()

    pltpu.sync_copy(agg_sh.at[pl.ds(r0, ROWS_PT)],
                    out_hbm.at[pl.ds(r0, ROWS_PT), pl.ds(c0, HALF)])


_agg = pl.kernel(
    _agg_body,
    out_type=jax.ShapeDtypeStruct((N, D), jnp.float32),
    mesh=plsc.VectorSubcoreMesh(core_axis_name="c", subcore_axis_name="s"),
    scratch_types=[
        pltpu.VMEM_SHARED((N, HALF), jnp.float32),
        pltpu.VMEM_SHARED((N + 8, HALF), jnp.float32),
        pltpu.VMEM((CPB, CHUNK), jnp.int32),
        pltpu.VMEM((CPB, CHUNK), jnp.int32),
        pltpu.VMEM((CHUNK, HALF), jnp.float32),
        pltpu.VMEM((CHUNK, HALF), jnp.float32),
        pltpu.VMEM((CHUNK, HALF), jnp.float32),
        pltpu.VMEM((CHUNK, HALF), jnp.float32),
        pltpu.SemaphoreType.DMA((NBUF,)),
        pltpu.SemaphoreType.DMA((NBUF,)),
    ],
    compiler_params=pltpu.CompilerParams(use_tc_tiling_on_sc=False),
)


def _mlp_body(z_ref, w_ref, b_ref, o_ref, *, relu):
    acc = jnp.dot(z_ref[...], w_ref[...],
                  preferred_element_type=jnp.float32) + b_ref[...]
    o_ref[...] = jnp.maximum(acc, 0.0) if relu else acc


def _mlp(z, w, b, relu):
    blk = 1000
    return pl.pallas_call(
        functools.partial(_mlp_body, relu=relu),
        grid=(N // blk,),
        in_specs=[
            pl.BlockSpec((blk, D), lambda i: (i, 0)),
            pl.BlockSpec((D, D), lambda i: (0, 0)),
            pl.BlockSpec((1, D), lambda i: (0, 0)),
        ],
        out_specs=pl.BlockSpec((blk, D), lambda i: (i, 0)),
        out_shape=jax.ShapeDtypeStruct((N, D), jnp.float32),
    )(z, w, b.reshape(1, D))


def kernel(x, edge_index, W1, b1, W2, b2, W3, b3):
    ei = edge_index.astype(jnp.int32)
    pad = E_PAD - E
    src = jnp.concatenate([ei[0], jnp.zeros((pad,), jnp.int32)])
    dst = jnp.concatenate([ei[1], jnp.full((pad,), N, jnp.int32)])
    src = src.reshape(NCHUNK, CHUNK)
    dst = dst.reshape(NCHUNK, CHUNK)

    h = x
    z = _agg(h, src, dst)
    h = _mlp(z, W1, b1, True)
    z = _agg(h, src, dst)
    h = _mlp(z, W2, b2, True)
    z = _agg(h, src, dst)
    return _mlp(z, W3, b3, False)


# speedup vs baseline: 1.1528x; 1.1528x over previous
import functools

import jax
import jax.numpy as jnp
from jax import lax
from jax.experimental import pallas as pl
from jax.experimental.pallas import tpu as pltpu
from jax.experimental.pallas import tpu_sc as plsc

N = 10000
D = 128
E = 320000
HALF = 64
NS = 16
CHUNK = 128
CPT = 160
NCHUNK = CPT * NS
E_PAD = NCHUNK * CHUNK
NBLK = 4
CPB = CPT // NBLK
ROWS_PT = N // NS
R_STEP = 125
R_ITER = ROWS_PT // R_STEP


NBUF = 4


def _agg_body(h_hbm, src_hbm, dst_hbm, out_hbm,
              h_sh, agg_sh, sidx_v, didx_v,
              buf0, buf1, buf2, buf3, gsems, ssems):
    c = lax.axis_index("c")
    s = lax.axis_index("s")
    c0 = c * HALF
    r0 = s * ROWS_PT
    bufs = (buf0, buf1, buf2, buf3)

    pltpu.sync_copy(h_hbm.at[pl.ds(r0, ROWS_PT), pl.ds(c0, HALF)],
                    h_sh.at[pl.ds(r0, ROWS_PT)])
    pltpu.sync_copy(h_hbm.at[pl.ds(r0, ROWS_PT), pl.ds(c0, HALF)],
                    agg_sh.at[pl.ds(r0, ROWS_PT)])
    plsc.subcore_barrier()

    def gather(j, t):
        return pltpu.async_copy(h_sh.at[sidx_v.at[j]], bufs[t], gsems.at[t])

    def gather_wait(j, t):
        pltpu.make_async_copy(h_sh.at[sidx_v.at[j]], bufs[t],
                              gsems.at[t]).wait()

    def scatter(j, t):
        return pltpu.async_copy(bufs[t], agg_sh.at[didx_v.at[j]],
                                ssems.at[t], add=True)

    def scatter_wait(j, t):
        pltpu.make_async_copy(bufs[t], agg_sh.at[didx_v.at[j]],
                              ssems.at[t]).wait()

    def blk(bi, carry):
        ch0 = s * CPT + bi * CPB
        pltpu.sync_copy(src_hbm.at[pl.ds(ch0, CPB)], sidx_v)
        pltpu.sync_copy(dst_hbm.at[pl.ds(ch0, CPB)], didx_v)

        def body(q, carry2):
            j = NBUF * q
            for t in range(NBUF):
                gather(j + t, t)
            for t in range(NBUF):
                gather_wait(j + t, t)
                scatter(j + t, t)
            for t in range(NBUF):
                scatter_wait(j + t, t)
            return carry2

        lax.fori_loop(0, CPB // NBUF, body, 0)
        return carry

    lax.fori_loop(0, NBLK, blk, 0)
    plsc.subcore_barrier()

    pltpu.sync_copy(agg_sh.at[pl.ds(r0, ROWS_PT)],
                    out_hbm.at[pl.ds(r0, ROWS_PT), pl.ds(c0, HALF)])


_agg = pl.kernel(
    _agg_body,
    out_type=jax.ShapeDtypeStruct((N, D), jnp.float32),
    mesh=plsc.VectorSubcoreMesh(core_axis_name="c", subcore_axis_name="s"),
    scratch_types=[
        pltpu.VMEM_SHARED((N, HALF), jnp.float32),
        pltpu.VMEM_SHARED((N + 8, HALF), jnp.float32),
        pltpu.VMEM((CPB, CHUNK), jnp.int32),
        pltpu.VMEM((CPB, CHUNK), jnp.int32),
        pltpu.VMEM((CHUNK, HALF), jnp.float32),
        pltpu.VMEM((CHUNK, HALF), jnp.float32),
        pltpu.VMEM((CHUNK, HALF), jnp.float32),
        pltpu.VMEM((CHUNK, HALF), jnp.float32),
        pltpu.SemaphoreType.DMA((NBUF,)),
        pltpu.SemaphoreType.DMA((NBUF,)),
    ],
    compiler_params=pltpu.CompilerParams(use_tc_tiling_on_sc=False),
)


def _mlp_body(z_ref, w_ref, b_ref, o_ref, *, relu):
    acc = jnp.dot(z_ref[...], w_ref[...],
                  preferred_element_type=jnp.float32) + b_ref[...]
    o_ref[...] = jnp.maximum(acc, 0.0) if relu else acc


def _mlp(z, w, b, relu):
    blk = 1000
    return pl.pallas_call(
        functools.partial(_mlp_body, relu=relu),
        grid=(N // blk,),
        in_specs=[
            pl.BlockSpec((blk, D), lambda i: (i, 0)),
            pl.BlockSpec((D, D), lambda i: (0, 0)),
            pl.BlockSpec((1, D), lambda i: (0, 0)),
        ],
        out_specs=pl.BlockSpec((blk, D), lambda i: (i, 0)),
        out_shape=jax.ShapeDtypeStruct((N, D), jnp.float32),
    )(z, w, b.reshape(1, D))


def kernel(x, edge_index, W1, b1, W2, b2, W3, b3):
    ei = edge_index.astype(jnp.int32)
    pad = E_PAD - E
    src = jnp.concatenate([ei[0], jnp.zeros((pad,), jnp.int32)])
    dst = jnp.concatenate([ei[1], jnp.full((pad,), N, jnp.int32)])
    src = src.reshape(NCHUNK, CHUNK)
    dst = dst.reshape(NCHUNK, CHUNK)

    h = x
    z = _agg(h, src, dst)
    h = _mlp(z, W1, b1, True)
    z = _agg(h, src, dst)
    h = _mlp(z, W2, b2, True)
    z = _agg(h, src, dst)
    return _mlp(z, W3, b3, False)
